# baseline (speedup 1.0000x reference)
import functools

import jax
import jax.numpy as jnp
import numpy as np
from jax import lax
from jax.experimental import pallas as pl
from jax.experimental.pallas import tpu as pltpu
from jax.experimental.pallas import tpu_sc as plsc

B = 4096
D = 64
VOCAB = 1000000
N_NEGS = 20

_info = plsc.get_sparse_core_info()
NC, NS, L = _info.num_cores, _info.num_subcores, _info.num_lanes
NW = NC * NS
BW = B // NW
NDOT = B * N_NEGS // NW
NDB = D // 8
WC = 2048


def _rotl(x, r):
    return ((x << np.uint32(r)) | (x >> np.uint32(32 - r))).astype(np.uint32)


def _threefry2x32(k1, k2, x0, x1):
    x0 = x0.astype(np.uint32).copy()
    x1 = x1.astype(np.uint32).copy()
    rot = (13, 15, 26, 6, 17, 29, 16, 24)
    ks = [np.uint32(k1), np.uint32(k2),
          np.uint32(np.uint32(k1) ^ np.uint32(k2) ^ np.uint32(0x1BD11BDA))]
    with np.errstate(over='ignore'):
        x0 = (x0 + ks[0]).astype(np.uint32)
        x1 = (x1 + ks[1]).astype(np.uint32)
        for base, ka, kb, inc in ((0, 1, 2, 1), (4, 2, 0, 2), (0, 0, 1, 3),
                                  (4, 1, 2, 4), (0, 2, 0, 5)):
            for i in range(4):
                x0 = (x0 + x1).astype(np.uint32)
                x1 = _rotl(x1, rot[base + i])
                x1 = (x0 ^ x1).astype(np.uint32)
            x0 = (x0 + ks[ka]).astype(np.uint32)
            x1 = (x1 + ks[kb] + np.uint32(inc)).astype(np.uint32)
    return x0, x1


def _np_randint(seed, n, span):
    k1 = np.uint32(np.uint64(seed) >> np.uint64(32))
    k2 = np.uint32(np.uint64(seed) & np.uint64(0xFFFFFFFF))
    s1, s2 = _threefry2x32(k1, k2, np.zeros(2, np.uint32),
                           np.arange(2, dtype=np.uint32))
    cz = np.zeros(n, np.uint32)
    ci = np.arange(n, dtype=np.uint32)
    h1, h2 = _threefry2x32(s1[0], s2[0], cz, ci)
    l1, l2 = _threefry2x32(s1[1], s2[1], cz, ci)
    hi, lo = (h1 ^ h2), (l1 ^ l2)
    span = np.uint32(span)
    with np.errstate(over='ignore'):
        mult = np.uint32(2 ** 16) % span
        mult = np.uint32((mult * mult) % span)
        off = ((hi % span) * mult + (lo % span)).astype(np.uint32) % span
    return off.astype(np.int32)


def _prep_tables():
    vocab_pad = ((VOCAB + 127) // 128) * 128
    v = _np_randint(42, B * N_NEGS, VOCAB)
    bb = (np.arange(B * N_NEGS, dtype=np.int64) // N_NEGS).astype(np.int32)
    order = np.argsort(v, kind='stable')
    sv, sb = v[order], bb[order]

    wins = []
    for w in range(NW):
        seg_v = sv[w * NDOT:(w + 1) * NDOT]
        seg_b = sb[w * NDOT:(w + 1) * NDOT]
        wlist = []
        i = 0
        while i < NDOT:
            lo = (int(seg_v[i]) // 128) * 128
            lo = min(lo, vocab_pad - WC)
            entries = []
            while i < NDOT and int(seg_v[i]) < lo + WC:
                entries.append((int(seg_v[i]) - lo, int(seg_b[i]), 1.0))
                i += 1
            while len(entries) % 16:
                entries.append((0, 0, 0.0))
            wlist.append((lo, entries))
        wins.append(wlist)

    nwmax = max(len(wl) for wl in wins)
    if nwmax % 2:
        nwmax += 1
    ndot_pad = max(sum(len(e) for _, e in wl) for wl in wins)

    vloc = np.zeros((NW, ndot_pad), np.int32)
    bidx = np.zeros((NW, ndot_pad), np.int32)
    wmask = np.zeros((NW, ndot_pad), np.float32)
    lotab = np.zeros((NW, nwmax), np.int32)
    gcum = np.zeros((NW, nwmax + 1), np.int32)
    for w, wl in enumerate(wins):
        pos = 0
        for i, (lo, entries) in enumerate(wl):
            lotab[w, i] = lo
            gcum[w, i] = pos // 16
            for (vl, bi, wt) in entries:
                vloc[w, pos] = vl
                bidx[w, pos] = bi
                wmask[w, pos] = wt
                pos += 1
        for i in range(len(wl), nwmax + 1):
            gcum[w, i] = pos // 16
    loexp = np.repeat(lotab.reshape(-1), 16).astype(np.int32)
    gcexp = np.repeat(gcum.reshape(-1), 16).astype(np.int32)
    return (vloc.reshape(-1), bidx.reshape(-1), wmask.reshape(-1),
            loexp, gcexp, nwmax, ndot_pad)


(_VLOC, _BIDX, _WMASK, _LOEXP, _GCEXP, _NWMAX, _NDOT_PAD) = _prep_tables()
_NGP = _NDOT_PAD // 16


def _logsig(x):
    a = jnp.exp(-jnp.abs(x))
    z = a / (a + 2.0)
    z2 = z * z
    p = 1.0 + z2 * (1.0 / 3 + z2 * (1.0 / 5 + z2 * (1.0 / 7 + z2 * (1.0 / 9))))
    return jnp.minimum(x, 0.0) - 2.0 * z * p


@functools.partial(
    pl.kernel,
    out_type=jax.ShapeDtypeStruct((NW * 16,), jnp.float32),
    mesh=plsc.VectorSubcoreMesh(core_axis_name="c", subcore_axis_name="s"),
    compiler_params=pltpu.CompilerParams(needs_layout_passes=False),
    scratch_types=[
        pltpu.VMEM((_NDOT_PAD,), jnp.int32),
        pltpu.VMEM((_NDOT_PAD,), jnp.int32),
        pltpu.VMEM((_NDOT_PAD,), jnp.float32),
        pltpu.VMEM((_NWMAX * 16,), jnp.int32),
        pltpu.VMEM(((_NWMAX + 1) * 16,), jnp.int32),
        pltpu.VMEM((_NDOT_PAD,), jnp.float32),
        pltpu.VMEM((BW,), jnp.float32),
        pltpu.VMEM((8, WC), jnp.float32),
        pltpu.VMEM((8, WC), jnp.float32),
        pltpu.VMEM((8, B), jnp.float32),
        pltpu.VMEM((8, B), jnp.float32),
        pltpu.VMEM_SHARED((D, B), jnp.float32),
        pltpu.VMEM((8, BW), jnp.float32),
        pltpu.VMEM((16,), jnp.float32),
        pltpu.SemaphoreType.DMA,
        pltpu.SemaphoreType.DMA,
        pltpu.SemaphoreType.DMA,
        pltpu.SemaphoreType.DMA,
    ],
)
def _sgns_sc(vloc_hbm, bidx_hbm, wmask_hbm, loexp_hbm, gcexp_hbm, embt_hbm,
             truet_hbm, outt_hbm, out_hbm,
             vloc_v, bidx_v, wmask_v, lo_v, gc_v, acc2_v, oacc_v,
             eblk0, eblk1, tblk0, tblk1, tshared, oblk, accv,
             esem0, esem1, tsem0, tsem1):
    wid = lax.axis_index("s") * NC + lax.axis_index("c")
    eblks = (eblk0, eblk1)
    esems = (esem0, esem1)
    tblks = (tblk0, tblk1)
    tsems = (tsem0, tsem1)

    pltpu.sync_copy(vloc_hbm.at[pl.ds(wid * _NDOT_PAD, _NDOT_PAD)], vloc_v)
    pltpu.sync_copy(bidx_hbm.at[pl.ds(wid * _NDOT_PAD, _NDOT_PAD)], bidx_v)
    pltpu.sync_copy(wmask_hbm.at[pl.ds(wid * _NDOT_PAD, _NDOT_PAD)], wmask_v)
    pltpu.sync_copy(loexp_hbm.at[pl.ds(wid * _NWMAX * 16, _NWMAX * 16)], lo_v)
    pltpu.sync_copy(
        gcexp_hbm.at[pl.ds(wid * (_NWMAX + 1) * 16, (_NWMAX + 1) * 16)], gc_v)

    def zero_body(i, _):
        acc2_v[pl.ds(i * 16, 16)] = jnp.zeros((16,), jnp.float32)
        return 0

    lax.fori_loop(0, _NGP, zero_body, 0)
    for g8 in range(BW // 16):
        oacc_v[pl.ds(g8 * 16, 16)] = jnp.zeros((16,), jnp.float32)

    def lo_at(wi):
        return pl.multiple_of(jnp.max(lo_v[pl.ds(wi * 16, 16)]), 128)

    def gc_at(i):
        return jnp.max(gc_v[pl.ds(i * 16, 16)])

    def emb_start(f, slot):
        dblk = f // _NWMAX
        wi = f % _NWMAX

        @pl.when(gc_at(wi + 1) > gc_at(wi))
        def _():
            lo = lo_at(wi)
            h = WC // 2
            src0 = embt_hbm.at[pl.ds(dblk * 8, 8), pl.ds(lo, h)]
            src1 = embt_hbm.at[pl.ds(dblk * 8, 8),
                               pl.ds(pl.multiple_of(lo + h, 128), h)]
            pltpu.make_async_copy(
                src0, eblks[slot].at[:, pl.ds(0, h)], esems[slot]).start()
            pltpu.make_async_copy(
                src1, eblks[slot].at[:, pl.ds(h, h)], esems[slot]).start()

    def emb_wait(wi, slot):
        @pl.when(gc_at(wi + 1) > gc_at(wi))
        def _():
            h = WC // 2
            src0 = embt_hbm.at[pl.ds(0, 8), pl.ds(0, h)]
            pltpu.make_async_copy(
                src0, eblks[slot].at[:, pl.ds(0, h)], esems[slot]).wait()
            pltpu.make_async_copy(
                src0, eblks[slot].at[:, pl.ds(h, h)], esems[slot]).wait()

    def tblk_start(dblk, par):
        src = tshared.at[pl.ds(dblk * 8, 8)]
        pltpu.make_async_copy(src, tblks[par], tsems[par]).start()

    def tblk_wait(par):
        src = tshared.at[pl.ds(0, 8)]
        pltpu.make_async_copy(src, tblks[par], tsems[par]).wait()

    @pl.when(lax.axis_index("s") == 0)
    def _():
        pltpu.sync_copy(truet_hbm, tshared)

    plsc.subcore_barrier()

    emb_start(0, 0)
    emb_start(1, 1)
    tblk_start(0, 0)

    rids = [jnp.full((16,), dp, jnp.int32) for dp in range(8)]

    def outer_body(p, _):
        for par in range(2):
            dblk = p * 2 + par
            tblk = tblks[par]
            tblk_wait(par)

            @pl.when(dblk + 1 < NDB)
            def _():
                tblk_start(dblk + 1, 1 - par)

            pltpu.sync_copy(
                outt_hbm.at[pl.ds(dblk * 8, 8), pl.ds(wid * BW, BW)], oblk)
            for g8 in range(BW // 16):
                a = oacc_v[pl.ds(g8 * 16, 16)]
                for dp in range(8):
                    a = a + (tblk[dp, pl.ds(wid * BW + g8 * 16, 16)]
                             * oblk[dp, pl.ds(g8 * 16, 16)])
                oacc_v[pl.ds(g8 * 16, 16)] = a

            def inner_body(q, _, dblk=dblk, tblk=tblk):
                for s2 in range(2):
                    wi = q * 2 + s2
                    emb_wait(wi, s2)
                    eblk = eblks[s2]

                    def g_body(g, _, eblk=eblk, tblk=tblk):
                        off = g * 16
                        vl = vloc_v[pl.ds(off, 16)]
                        bl = bidx_v[pl.ds(off, 16)]
                        a = acc2_v[pl.ds(off, 16)]
                        for dp in range(8):
                            e = plsc.load_gather(eblk, [rids[dp], vl])
                            t = plsc.load_gather(tblk, [rids[dp], bl])
                            a = a + e * t
                        acc2_v[pl.ds(off, 16)] = a
                        return 0

                    lax.fori_loop(gc_at(wi), gc_at(wi + 1), g_body, 0)
                    f_next = dblk * _NWMAX + wi + 2

                    @pl.when(f_next < NDB * _NWMAX)
                    def _():
                        emb_start(f_next, s2)
                return 0

            lax.fori_loop(0, _NWMAX // 2, inner_body, 0)
        return 0

    lax.fori_loop(0, NDB // 2, outer_body, 0)

    def n_body(g, acc):
        dv = acc2_v[pl.ds(g * 16, 16)]
        w = wmask_v[pl.ds(g * 16, 16)]
        return acc + _logsig(-dv) * w

    acc = lax.fori_loop(0, _NGP, n_body, jnp.zeros((16,), jnp.float32))
    for g8 in range(BW // 16):
        acc = acc + _logsig(oacc_v[pl.ds(g8 * 16, 16)])

    accv[...] = acc
    pltpu.sync_copy(accv, out_hbm.at[pl.ds(wid * 16, 16)])


def kernel(true_vecs, out_vecs, emb_table):
    embt = emb_table.T
    truet = true_vecs.reshape(B, D).T
    outt = out_vecs.reshape(B, D).T
    partials = _sgns_sc(jnp.asarray(_VLOC), jnp.asarray(_BIDX),
                        jnp.asarray(_WMASK), jnp.asarray(_LOEXP),
                        jnp.asarray(_GCEXP), embt, truet, outt)
    return -(jnp.sum(partials) / jnp.float32(B))


# speedup vs baseline: 1.0017x; 1.0017x over previous
import functools

import jax
import jax.numpy as jnp
import numpy as np
from jax import lax
from jax.experimental import pallas as pl
from jax.experimental.pallas import tpu as pltpu
from jax.experimental.pallas import tpu_sc as plsc

B = 4096
D = 64
VOCAB = 1000000
N_NEGS = 20

_info = plsc.get_sparse_core_info()
NC, NS, L = _info.num_cores, _info.num_subcores, _info.num_lanes
NW = NC * NS
BW = B // NW
NDOT = B * N_NEGS // NW
NDB = D // 8
WC = 2048


def _rotl(x, r):
    return ((x << np.uint32(r)) | (x >> np.uint32(32 - r))).astype(np.uint32)


def _threefry2x32(k1, k2, x0, x1):
    x0 = x0.astype(np.uint32).copy()
    x1 = x1.astype(np.uint32).copy()
    rot = (13, 15, 26, 6, 17, 29, 16, 24)
    ks = [np.uint32(k1), np.uint32(k2),
          np.uint32(np.uint32(k1) ^ np.uint32(k2) ^ np.uint32(0x1BD11BDA))]
    with np.errstate(over='ignore'):
        x0 = (x0 + ks[0]).astype(np.uint32)
        x1 = (x1 + ks[1]).astype(np.uint32)
        for base, ka, kb, inc in ((0, 1, 2, 1), (4, 2, 0, 2), (0, 0, 1, 3),
                                  (4, 1, 2, 4), (0, 2, 0, 5)):
            for i in range(4):
                x0 = (x0 + x1).astype(np.uint32)
                x1 = _rotl(x1, rot[base + i])
                x1 = (x0 ^ x1).astype(np.uint32)
            x0 = (x0 + ks[ka]).astype(np.uint32)
            x1 = (x1 + ks[kb] + np.uint32(inc)).astype(np.uint32)
    return x0, x1


def _np_randint(seed, n, span):
    k1 = np.uint32(np.uint64(seed) >> np.uint64(32))
    k2 = np.uint32(np.uint64(seed) & np.uint64(0xFFFFFFFF))
    s1, s2 = _threefry2x32(k1, k2, np.zeros(2, np.uint32),
                           np.arange(2, dtype=np.uint32))
    cz = np.zeros(n, np.uint32)
    ci = np.arange(n, dtype=np.uint32)
    h1, h2 = _threefry2x32(s1[0], s2[0], cz, ci)
    l1, l2 = _threefry2x32(s1[1], s2[1], cz, ci)
    hi, lo = (h1 ^ h2), (l1 ^ l2)
    span = np.uint32(span)
    with np.errstate(over='ignore'):
        mult = np.uint32(2 ** 16) % span
        mult = np.uint32((mult * mult) % span)
        off = ((hi % span) * mult + (lo % span)).astype(np.uint32) % span
    return off.astype(np.int32)


def _prep_tables():
    vocab_pad = ((VOCAB + 127) // 128) * 128
    v = _np_randint(42, B * N_NEGS, VOCAB)
    bb = (np.arange(B * N_NEGS, dtype=np.int64) // N_NEGS).astype(np.int32)
    order = np.argsort(v, kind='stable')
    sv, sb = v[order], bb[order]

    wins = []
    for w in range(NW):
        seg_v = sv[w * NDOT:(w + 1) * NDOT]
        seg_b = sb[w * NDOT:(w + 1) * NDOT]
        wlist = []
        i = 0
        while i < NDOT:
            lo = (int(seg_v[i]) // 128) * 128
            lo = min(lo, vocab_pad - WC)
            entries = []
            while i < NDOT and int(seg_v[i]) < lo + WC:
                entries.append((int(seg_v[i]) - lo, int(seg_b[i]), 1.0))
                i += 1
            while len(entries) % 16:
                entries.append((0, 0, 0.0))
            wlist.append((lo, entries))
        wins.append(wlist)

    nwmax = max(len(wl) for wl in wins)
    if nwmax % 2:
        nwmax += 1
    ndot_pad = max(sum(len(e) for _, e in wl) for wl in wins)

    vloc = np.zeros((NW, ndot_pad), np.int32)
    bidx = np.zeros((NW, ndot_pad), np.int32)
    wmask = np.zeros((NW, ndot_pad), np.float32)
    lotab = np.zeros((NW, nwmax), np.int32)
    gcum = np.zeros((NW, nwmax + 1), np.int32)
    for w, wl in enumerate(wins):
        pos = 0
        for i, (lo, entries) in enumerate(wl):
            lotab[w, i] = lo
            gcum[w, i] = pos // 16
            for (vl, bi, wt) in entries:
                vloc[w, pos] = vl
                bidx[w, pos] = bi
                wmask[w, pos] = wt
                pos += 1
        for i in range(len(wl), nwmax + 1):
            gcum[w, i] = pos // 16
    loexp = np.repeat(lotab.reshape(-1), 16).astype(np.int32)
    gcexp = np.repeat(gcum.reshape(-1), 16).astype(np.int32)
    return (vloc.reshape(-1), bidx.reshape(-1), wmask.reshape(-1),
            loexp, gcexp, nwmax, ndot_pad)


(_VLOC, _BIDX, _WMASK, _LOEXP, _GCEXP, _NWMAX, _NDOT_PAD) = _prep_tables()
_NGP = _NDOT_PAD // 16


def _logsig(x):
    a = jnp.exp(-jnp.abs(x))
    z = a / (a + 2.0)
    z2 = z * z
    p = 1.0 + z2 * (1.0 / 3 + z2 * (1.0 / 5 + z2 * (1.0 / 7 + z2 * (1.0 / 9))))
    return jnp.minimum(x, 0.0) - 2.0 * z * p


@functools.partial(
    pl.kernel,
    out_type=jax.ShapeDtypeStruct((NW * 16,), jnp.float32),
    mesh=plsc.VectorSubcoreMesh(core_axis_name="c", subcore_axis_name="s"),
    compiler_params=pltpu.CompilerParams(needs_layout_passes=False),
    scratch_types=[
        pltpu.VMEM((_NDOT_PAD,), jnp.int32),
        pltpu.VMEM((_NDOT_PAD,), jnp.int32),
        pltpu.VMEM((_NDOT_PAD,), jnp.float32),
        pltpu.VMEM((_NWMAX * 16,), jnp.int32),
        pltpu.VMEM(((_NWMAX + 1) * 16,), jnp.int32),
        pltpu.VMEM((_NDOT_PAD,), jnp.float32),
        pltpu.VMEM((BW,), jnp.float32),
        pltpu.VMEM((8, WC), jnp.float32),
        pltpu.VMEM((8, WC), jnp.float32),
        pltpu.VMEM((8, B), jnp.float32),
        pltpu.VMEM((8, B), jnp.float32),
        pltpu.VMEM_SHARED((D, B), jnp.float32),
        pltpu.VMEM((8, BW), jnp.float32),
        pltpu.VMEM((16,), jnp.float32),
        pltpu.SemaphoreType.DMA,
        pltpu.SemaphoreType.DMA,
        pltpu.SemaphoreType.DMA,
        pltpu.SemaphoreType.DMA,
    ],
)
def _sgns_sc(vloc_hbm, bidx_hbm, wmask_hbm, loexp_hbm, gcexp_hbm, embt_hbm,
             truet_hbm, outt_hbm, out_hbm,
             vloc_v, bidx_v, wmask_v, lo_v, gc_v, acc2_v, oacc_v,
             eblk0, eblk1, tblk0, tblk1, tshared, oblk, accv,
             esem0, esem1, tsem0, tsem1):
    wid = lax.axis_index("s") * NC + lax.axis_index("c")
    eblks = (eblk0, eblk1)
    esems = (esem0, esem1)
    tblks = (tblk0, tblk1)
    tsems = (tsem0, tsem1)

    pltpu.sync_copy(vloc_hbm.at[pl.ds(wid * _NDOT_PAD, _NDOT_PAD)], vloc_v)
    pltpu.sync_copy(bidx_hbm.at[pl.ds(wid * _NDOT_PAD, _NDOT_PAD)], bidx_v)
    pltpu.sync_copy(wmask_hbm.at[pl.ds(wid * _NDOT_PAD, _NDOT_PAD)], wmask_v)
    pltpu.sync_copy(loexp_hbm.at[pl.ds(wid * _NWMAX * 16, _NWMAX * 16)], lo_v)
    pltpu.sync_copy(
        gcexp_hbm.at[pl.ds(wid * (_NWMAX + 1) * 16, (_NWMAX + 1) * 16)], gc_v)

    def zero_body(i, _):
        acc2_v[pl.ds(i * 16, 16)] = jnp.zeros((16,), jnp.float32)
        return 0

    lax.fori_loop(0, _NGP, zero_body, 0)
    for g8 in range(BW // 16):
        oacc_v[pl.ds(g8 * 16, 16)] = jnp.zeros((16,), jnp.float32)

    def lo_at(wi):
        return pl.multiple_of(jnp.max(lo_v[pl.ds(wi * 16, 16)]), 128)

    def gc_at(i):
        return jnp.max(gc_v[pl.ds(i * 16, 16)])

    def emb_start(f, slot):
        dblk = f // _NWMAX
        wi = f % _NWMAX

        @pl.when(gc_at(wi + 1) > gc_at(wi))
        def _():
            src = embt_hbm.at[pl.ds(dblk * 8, 8), pl.ds(lo_at(wi), WC)]
            pltpu.make_async_copy(src, eblks[slot], esems[slot]).start()

    def emb_wait(wi, slot):
        @pl.when(gc_at(wi + 1) > gc_at(wi))
        def _():
            src = embt_hbm.at[pl.ds(0, 8), pl.ds(0, WC)]
            pltpu.make_async_copy(src, eblks[slot], esems[slot]).wait()

    def tblk_start(dblk, par):
        src = tshared.at[pl.ds(dblk * 8, 8)]
        pltpu.make_async_copy(src, tblks[par], tsems[par]).start()

    def tblk_wait(par):
        src = tshared.at[pl.ds(0, 8)]
        pltpu.make_async_copy(src, tblks[par], tsems[par]).wait()

    @pl.when(lax.axis_index("s") == 0)
    def _():
        pltpu.sync_copy(truet_hbm, tshared)

    plsc.subcore_barrier()

    emb_start(0, 0)
    emb_start(1, 1)
    tblk_start(0, 0)

    rids = [jnp.full((16,), dp, jnp.int32) for dp in range(8)]

    def outer_body(p, _):
        for par in range(2):
            dblk = p * 2 + par
            tblk = tblks[par]
            tblk_wait(par)

            @pl.when(dblk + 1 < NDB)
            def _():
                tblk_start(dblk + 1, 1 - par)

            pltpu.sync_copy(
                outt_hbm.at[pl.ds(dblk * 8, 8), pl.ds(wid * BW, BW)], oblk)
            for g8 in range(BW // 16):
                a = oacc_v[pl.ds(g8 * 16, 16)]
                for dp in range(8):
                    a = a + (tblk[dp, pl.ds(wid * BW + g8 * 16, 16)]
                             * oblk[dp, pl.ds(g8 * 16, 16)])
                oacc_v[pl.ds(g8 * 16, 16)] = a

            def inner_body(q, _, dblk=dblk, tblk=tblk):
                for s2 in range(2):
                    wi = q * 2 + s2
                    emb_wait(wi, s2)
                    eblk = eblks[s2]

                    def g_body(g, _, eblk=eblk, tblk=tblk):
                        off = g * 16
                        vl = vloc_v[pl.ds(off, 16)]
                        bl = bidx_v[pl.ds(off, 16)]
                        a = acc2_v[pl.ds(off, 16)]
                        for dp in range(8):
                            e = plsc.load_gather(eblk, [rids[dp], vl])
                            t = plsc.load_gather(tblk, [rids[dp], bl])
                            a = a + e * t
                        acc2_v[pl.ds(off, 16)] = a
                        return 0

                    lax.fori_loop(gc_at(wi), gc_at(wi + 1), g_body, 0)
                    f_next = dblk * _NWMAX + wi + 2

                    @pl.when(f_next < NDB * _NWMAX)
                    def _():
                        emb_start(f_next, s2)
                return 0

            lax.fori_loop(0, _NWMAX // 2, inner_body, 0)
        return 0

    lax.fori_loop(0, NDB // 2, outer_body, 0)

    def n_body(g, acc):
        dv = acc2_v[pl.ds(g * 16, 16)]
        w = wmask_v[pl.ds(g * 16, 16)]
        return acc + _logsig(-dv) * w

    acc = lax.fori_loop(0, _NGP, n_body, jnp.zeros((16,), jnp.float32))
    for g8 in range(BW // 16):
        acc = acc + _logsig(oacc_v[pl.ds(g8 * 16, 16)])

    accv[...] = acc
    pltpu.sync_copy(accv, out_hbm.at[pl.ds(wid * 16, 16)])


def kernel(true_vecs, out_vecs, emb_table):
    embt = emb_table.T
    truet = true_vecs.reshape(B, D).T
    outt = out_vecs.reshape(B, D).T
    partials = _sgns_sc(jnp.asarray(_VLOC), jnp.asarray(_BIDX),
                        jnp.asarray(_WMASK), jnp.asarray(_LOEXP),
                        jnp.asarray(_GCEXP), embt, truet, outt)
    return -(jnp.sum(partials) / jnp.float32(B))
